# baseline (speedup 1.0000x reference)
import functools

import jax
import jax.numpy as jnp
from jax import lax
from jax.experimental import pallas as pl
from jax.experimental.pallas import tpu as pltpu
from jax.experimental.pallas import tpu_sc as plsc

B, S, N, D, H, NI = 512, 50, 12, 64, 128, 1000000
ROWS = B * S
NW = 32
RPW = ROWS // NW
CH = 80
NCH = RPW // CH


def _sc_gather(items_flat, neighbor_table, nwt, item_table):
    mesh = plsc.VectorSubcoreMesh(core_axis_name="c", subcore_axis_name="s")

    @functools.partial(
        pl.kernel,
        out_type=(
            jax.ShapeDtypeStruct((ROWS, D), jnp.float32),
            jax.ShapeDtypeStruct((ROWS, 16), jnp.float32),
            jax.ShapeDtypeStruct((N * ROWS, D), jnp.float32),
        ),
        mesh=mesh,
        compiler_params=pltpu.CompilerParams(use_tc_tiling_on_sc=False),
        scratch_types=[
            pltpu.VMEM((RPW,), jnp.int32),
            pltpu.VMEM((N * RPW,), jnp.int32),
            pltpu.VMEM((N * RPW,), jnp.int32),
            pltpu.VMEM((RPW, 16), jnp.float32),
            pltpu.VMEM((RPW // 2, D), jnp.float32),
            pltpu.VMEM((RPW // 2, D), jnp.float32),
            pltpu.SemaphoreType.DMA,
            pltpu.SemaphoreType.DMA,
            pltpu.SemaphoreType.DMA,
            pltpu.SemaphoreType.DMA,
        ],
    )
    def sc_kernel(items_hbm, nbtf_hbm, nwt16_hbm, itt_hbm,
                  hid_out, nbw_out, nbf_out,
                  items_v, addr_v, idx_t, wgt_v, buf_a, buf_b,
                  sem_a, sem_b, sem_c, sem_e):
        wid = lax.axis_index("s") * 2 + lax.axis_index("c")
        base = wid * RPW
        pltpu.sync_copy(items_hbm.at[pl.ds(base, RPW)], items_v)

        hid_d, wgt_d = [], []
        for c in range(NCH):
            idx_c = items_v.at[pl.ds(c * CH, CH)]
            buf = buf_a if c < NCH // 2 else buf_b
            hid_d.append(pltpu.async_copy(
                itt_hbm.at[idx_c],
                buf.at[pl.ds((c % (NCH // 2)) * CH, CH)], sem_b))
            wgt_d.append(pltpu.async_copy(
                nwt16_hbm.at[idx_c], wgt_v.at[pl.ds(c * CH, CH)], sem_c))

        def addr_body(k, _):
            v = items_v[pl.ds(k * 16, 16)] * 16
            for n in range(N):
                addr_v[pl.ds(n * RPW + k * 16, 16)] = v + n
            return _
        lax.fori_loop(0, RPW // 16, addr_body, None)

        nbs_d = []
        for t in range(N * NCH):
            off = t * CH
            nbs_d.append(pltpu.async_copy(
                nbtf_hbm.at[addr_v.at[pl.ds(off, CH)]],
                idx_t.at[pl.ds(off, CH)], sem_a))

        for d in hid_d:
            d.wait()
        pltpu.sync_copy(buf_a, hid_out.at[pl.ds(base, RPW // 2)])
        pltpu.sync_copy(buf_b, hid_out.at[pl.ds(base + RPW // 2, RPW // 2)])
        for d in wgt_d:
            d.wait()
        pltpu.sync_copy(wgt_v, nbw_out.at[pl.ds(base, RPW)])
        for d in nbs_d:
            d.wait()

        HCH = NCH // 2
        HR = RPW // 2
        prev2 = None
        for t in range(2 * N):
            n, half = t // 2, t % 2
            buf = buf_a if t % 2 == 0 else buf_b
            sem_t = sem_a if t % 2 == 0 else sem_e
            o = n * RPW + half * HR
            descs = [pltpu.async_copy(
                itt_hbm.at[idx_t.at[pl.ds(o + c * CH, CH)]],
                buf.at[pl.ds(c * CH, CH)], sem_t) for c in range(HCH)]
            if prev2 is not None:
                pt, pbuf, pdescs = prev2
                for d in pdescs:
                    d.wait()
                po = (pt // 2) * ROWS + base + (pt % 2) * HR
                pltpu.sync_copy(pbuf, nbf_out.at[pl.ds(po, HR)])
            prev2 = (t, buf, descs)
        pt, pbuf, pdescs = prev2
        for d in pdescs:
            d.wait()
        po = (pt // 2) * ROWS + base + (pt % 2) * HR
        pltpu.sync_copy(pbuf, nbf_out.at[pl.ds(po, HR)])

    nbt16f = jnp.pad(neighbor_table, ((0, 0), (0, 4))).reshape((NI + 1) * 16)
    nwt16 = jnp.pad(nwt, ((0, 0), (0, 4)))
    return sc_kernel(items_flat, nbt16f, nwt16, item_table)


R = 800
GRID = ROWS // R


def _tc_body(nbf_ref, hid_ref, nbw_ref, items_ref, sess_ref, pos_ref,
             w1t_ref, w1b_ref, b1_ref, w2both_ref, b2cat_ref, exp_ref,
             woa0_ref, wob0_ref, bo0_ref, g0_ref, be0_ref,
             woa1_ref, wob1_ref, bo1_ref, g1_ref, be1_ref,
             hidden_out_ref, aw0_out_ref, aw1_out_ref):
    f32 = jnp.float32
    bf16 = jnp.bfloat16
    sess_e = sess_ref[...]
    sproj = jnp.dot(sess_e, w1t_ref[...],
                    preferred_element_type=f32) + b1_ref[...]
    nbf = nbf_ref[...]
    nbf_bf = nbf.astype(bf16)
    w1b = w1b_ref[...]
    w2both = w2both_ref[...]
    sboth = jnp.zeros((R, 2 * N), f32)
    for n in range(N):
        hp = jnp.dot(nbf_bf[n], w1b, preferred_element_type=f32) + sproj
        h = jnp.tanh(hp)
        sboth = sboth + jnp.dot(h.astype(bf16), w2both[n],
                                preferred_element_type=f32)
    sboth = sboth + b2cat_ref[...]
    s0 = sboth[:, :N]
    s1 = sboth[:, N:]
    valid = items_ref[...] > 0
    w = jnp.where(valid, nbw_ref[...][:, :N], 0.0)
    mask = w > 0
    s0 = jnp.where(mask, s0 * w, -1e9)
    s1 = jnp.where(mask, s1 * w, -1e9)

    def _softmax(s):
        m = jnp.max(s, axis=1, keepdims=True)
        e = jnp.exp(s - m)
        return e / jnp.sum(e, axis=1, keepdims=True)

    aw0 = _softmax(s0)
    aw1 = _softmax(s1)
    aw0_out_ref[...] = aw0
    aw1_out_ref[...] = aw1

    exp = exp_ref[...]
    aw0e = jnp.dot(aw0, exp, preferred_element_type=f32)
    aw1e = jnp.dot(aw1, exp, preferred_element_type=f32)
    ctx0 = jnp.zeros((R, D), f32)
    ctx1 = jnp.zeros((R, D), f32)
    for n in range(N):
        ctx0 = ctx0 + aw0e[:, n * D:(n + 1) * D] * nbf[n]
        ctx1 = ctx1 + aw1e[:, n * D:(n + 1) * D] * nbf[n]
    vf = valid.astype(f32)
    ctx0 = ctx0 * vf
    ctx1 = ctx1 * vf

    hidden = hid_ref[...] + pos_ref[...]
    for ctx, woa, wob, bo, g, be in (
            (ctx0, woa0_ref, wob0_ref, bo0_ref, g0_ref, be0_ref),
            (ctx1, woa1_ref, wob1_ref, bo1_ref, g1_ref, be1_ref)):
        out = (jnp.dot(hidden, woa[...], preferred_element_type=f32)
               + jnp.dot(ctx, wob[...], preferred_element_type=f32)
               + bo[...])
        x = hidden + out
        mu = jnp.mean(x, axis=1, keepdims=True)
        xc = x - mu
        var = jnp.mean(xc * xc, axis=1, keepdims=True)
        hidden = xc * lax.rsqrt(var + 1e-5) * g[...] + be[...]
    hidden_out_ref[...] = hidden


def _tc_forward(nbf_t3, hid_rows, nbw_rows, items_col, sess, pos_table,
                w1t, w1b, b1r, w2both, b2cat, exp_mat,
                woa0, wob0, bo0, g0, be0, woa1, wob1, bo1, g1, be1,
                interpret=False):
    full = lambda shape: pl.BlockSpec(shape, lambda i: tuple(0 for _ in shape))
    return pl.pallas_call(
        _tc_body,
        grid=(GRID,),
        in_specs=[
            pl.BlockSpec((N, R, D), lambda i: (0, i, 0)),
            pl.BlockSpec((R, D), lambda i: (i, 0)),
            pl.BlockSpec((R, 16), lambda i: (i, 0)),
            pl.BlockSpec((R, 1), lambda i: (i, 0)),
            pl.BlockSpec((R, D), lambda i: (i, 0)),
            pl.BlockSpec((R, D), lambda i: (i, 0)),
            full((D, 2 * H)),
            full((D, 2 * H)),
            full((1, 2 * H)),
            full((N, 2 * H, 2 * N)),
            full((1, 2 * N)),
            full((N, N * D)),
            full((D, D)), full((D, D)), full((1, D)), full((1, D)), full((1, D)),
            full((D, D)), full((D, D)), full((1, D)), full((1, D)), full((1, D)),
        ],
        out_specs=[
            pl.BlockSpec((R, D), lambda i: (i, 0)),
            pl.BlockSpec((R, N), lambda i: (i, 0)),
            pl.BlockSpec((R, N), lambda i: (i, 0)),
        ],
        out_shape=[
            jax.ShapeDtypeStruct((ROWS, D), jnp.float32),
            jax.ShapeDtypeStruct((ROWS, N), jnp.float32),
            jax.ShapeDtypeStruct((ROWS, N), jnp.float32),
        ],
        interpret=interpret,
    )(nbf_t3, hid_rows, nbw_rows, items_col, sess, pos_table,
      w1t, w1b, b1r, w2both, b2cat, exp_mat,
      woa0, wob0, bo0, g0, be0, woa1, wob1, bo1, g1, be1)


def kernel(items, neighbor_table, neighbor_weight_table, session_embedding,
           item_table, pos_table,
           W1_0, b1_0, W2_0, b2_0, Wo_0, bo_0, g_0, be_0,
           W1_1, b1_1, W2_1, b2_1, Wo_1, bo_1, g_1, be_1):
    items_flat = items.reshape(ROWS).astype(jnp.int32)
    hid_rows, nbw_rows, nbf_t = _sc_gather(
        items_flat, neighbor_table, neighbor_weight_table, item_table)
    nbf_t3 = nbf_t.reshape(N, ROWS, D)

    w1t = jnp.concatenate([W1_0[:D], W1_1[:D]], axis=1)
    w1b = jnp.concatenate([W1_0[D:], W1_1[D:]],
                          axis=1).astype(jnp.bfloat16)
    b1r = jnp.concatenate([b1_0, b1_1], axis=0).reshape(1, 2 * H)
    eye = jnp.eye(N, dtype=jnp.float32)
    z = jnp.zeros((N, H, N), jnp.float32)
    top = jnp.concatenate(
        [W2_0.reshape(1, H, 1) * eye[:, None, :], z], axis=2)
    bot = jnp.concatenate(
        [z, W2_1.reshape(1, H, 1) * eye[:, None, :]], axis=2)
    w2both = jnp.concatenate([top, bot], axis=1).astype(jnp.bfloat16)
    b2cat = jnp.concatenate(
        [jnp.broadcast_to(b2_0, (N,)), jnp.broadcast_to(b2_1, (N,))],
        axis=0).reshape(1, 2 * N)
    exp_mat = jnp.repeat(jnp.eye(N, dtype=jnp.float32), D, axis=1)

    sess_exp = jnp.broadcast_to(
        session_embedding[:, None, :], (B, S, D)).reshape(ROWS, D)
    pos_exp = jnp.broadcast_to(
        pos_table[None, :S, :], (B, S, D)).reshape(ROWS, D)
    hidden_f, aw0_f, aw1_f = _tc_forward(
        nbf_t3, hid_rows, nbw_rows, items_flat.reshape(ROWS, 1),
        sess_exp, pos_exp,
        w1t, w1b, b1r, w2both, b2cat, exp_mat,
        Wo_0[:D], Wo_0[D:], bo_0.reshape(1, D), g_0.reshape(1, D),
        be_0.reshape(1, D),
        Wo_1[:D], Wo_1[D:], bo_1.reshape(1, D), g_1.reshape(1, D),
        be_1.reshape(1, D))

    return (hidden_f.reshape(B, S, D),
            aw0_f.reshape(B, S, N),
            aw1_f.reshape(B, S, N))


# speedup vs baseline: 2.1089x; 2.1089x over previous
import functools

import jax
import jax.numpy as jnp
from jax import lax
from jax.experimental import pallas as pl
from jax.experimental.pallas import tpu as pltpu
from jax.experimental.pallas import tpu_sc as plsc

B, S, N, D, H, NI = 512, 50, 12, 64, 128, 1000000
ROWS = B * S
NW = 32
RPW = ROWS // NW
CH = 80
NCH = RPW // CH


def _sc_gather(items_flat, nbs_t_flat, item_table):
    mesh = plsc.VectorSubcoreMesh(core_axis_name="c", subcore_axis_name="s")

    @functools.partial(
        pl.kernel,
        out_type=(
            jax.ShapeDtypeStruct((ROWS, D), jnp.float32),
            jax.ShapeDtypeStruct((N * ROWS, D), jnp.float32),
        ),
        mesh=mesh,
        compiler_params=pltpu.CompilerParams(use_tc_tiling_on_sc=False),
        scratch_types=[
            pltpu.VMEM((RPW,), jnp.int32),
            pltpu.VMEM((N * RPW,), jnp.int32),
            pltpu.VMEM((RPW // 2, D), jnp.float32),
            pltpu.VMEM((RPW // 2, D), jnp.float32),
            pltpu.SemaphoreType.DMA,
            pltpu.SemaphoreType.DMA,
            pltpu.SemaphoreType.DMA,
        ],
    )
    def sc_kernel(items_hbm, nbst_hbm, itt_hbm,
                  hid_out, nbf_out,
                  items_v, idx_t, buf_a, buf_b,
                  sem_a, sem_b, sem_e):
        wid = lax.axis_index("s") * 2 + lax.axis_index("c")
        base = wid * RPW
        pltpu.sync_copy(items_hbm.at[pl.ds(base, RPW)], items_v)

        idx_d = [pltpu.async_copy(
            nbst_hbm.at[pl.ds(n * ROWS + base, RPW)],
            idx_t.at[pl.ds(n * RPW, RPW)], sem_a) for n in range(N)]

        hid_d = []
        for c in range(NCH):
            idx_c = items_v.at[pl.ds(c * CH, CH)]
            buf = buf_a if c < NCH // 2 else buf_b
            hid_d.append(pltpu.async_copy(
                itt_hbm.at[idx_c],
                buf.at[pl.ds((c % (NCH // 2)) * CH, CH)], sem_b))

        for d in hid_d:
            d.wait()
        pltpu.sync_copy(buf_a, hid_out.at[pl.ds(base, RPW // 2)])
        pltpu.sync_copy(buf_b, hid_out.at[pl.ds(base + RPW // 2, RPW // 2)])
        for d in idx_d:
            d.wait()

        HCH = NCH // 2
        HR = RPW // 2
        prev2 = None
        for t in range(2 * N):
            n, half = t // 2, t % 2
            buf = buf_a if t % 2 == 0 else buf_b
            sem_t = sem_a if t % 2 == 0 else sem_e
            o = n * RPW + half * HR
            descs = [pltpu.async_copy(
                itt_hbm.at[idx_t.at[pl.ds(o + c * CH, CH)]],
                buf.at[pl.ds(c * CH, CH)], sem_t) for c in range(HCH)]
            if prev2 is not None:
                pt, pbuf, pdescs = prev2
                for d in pdescs:
                    d.wait()
                po = (pt // 2) * ROWS + base + (pt % 2) * HR
                pltpu.sync_copy(pbuf, nbf_out.at[pl.ds(po, HR)])
            prev2 = (t, buf, descs)
        pt, pbuf, pdescs = prev2
        for d in pdescs:
            d.wait()
        po = (pt // 2) * ROWS + base + (pt % 2) * HR
        pltpu.sync_copy(pbuf, nbf_out.at[pl.ds(po, HR)])

    return sc_kernel(items_flat, nbs_t_flat, item_table)


R = 800
GRID = ROWS // R


def _tc_body(nbf_ref, hid_ref, nbw_ref, items_ref, sess_ref, pos_ref,
             w1t_ref, w1b_ref, b1_ref, w2both_ref, b2cat_ref, exp_ref,
             woa0_ref, wob0_ref, bo0_ref, g0_ref, be0_ref,
             woa1_ref, wob1_ref, bo1_ref, g1_ref, be1_ref,
             hidden_out_ref, aw0_out_ref, aw1_out_ref):
    f32 = jnp.float32
    bf16 = jnp.bfloat16
    sess_e = sess_ref[...]
    sproj = jnp.dot(sess_e, w1t_ref[...],
                    preferred_element_type=f32) + b1_ref[...]
    nbf = nbf_ref[...]
    nbf_bf = nbf.astype(bf16)
    w1b = w1b_ref[...]
    w2both = w2both_ref[...]
    sboth = jnp.zeros((R, 2 * N), f32)
    for n in range(N):
        hp = jnp.dot(nbf_bf[n], w1b, preferred_element_type=f32) + sproj
        h = jnp.tanh(hp)
        sboth = sboth + jnp.dot(h.astype(bf16), w2both[n],
                                preferred_element_type=f32)
    sboth = sboth + b2cat_ref[...]
    s0 = sboth[:, :N]
    s1 = sboth[:, N:]
    valid = items_ref[...] > 0
    w = jnp.where(valid, nbw_ref[...], 0.0)
    mask = w > 0
    s0 = jnp.where(mask, s0 * w, -1e9)
    s1 = jnp.where(mask, s1 * w, -1e9)

    def _softmax(s):
        m = jnp.max(s, axis=1, keepdims=True)
        e = jnp.exp(s - m)
        return e / jnp.sum(e, axis=1, keepdims=True)

    aw0 = _softmax(s0)
    aw1 = _softmax(s1)
    aw0_out_ref[...] = aw0
    aw1_out_ref[...] = aw1

    exp = exp_ref[...]
    aw0e = jnp.dot(aw0, exp, preferred_element_type=f32)
    aw1e = jnp.dot(aw1, exp, preferred_element_type=f32)
    ctx0 = jnp.zeros((R, D), f32)
    ctx1 = jnp.zeros((R, D), f32)
    for n in range(N):
        ctx0 = ctx0 + aw0e[:, n * D:(n + 1) * D] * nbf[n]
        ctx1 = ctx1 + aw1e[:, n * D:(n + 1) * D] * nbf[n]
    vf = valid.astype(f32)
    ctx0 = ctx0 * vf
    ctx1 = ctx1 * vf

    hidden = hid_ref[...] + pos_ref[...]
    for ctx, woa, wob, bo, g, be in (
            (ctx0, woa0_ref, wob0_ref, bo0_ref, g0_ref, be0_ref),
            (ctx1, woa1_ref, wob1_ref, bo1_ref, g1_ref, be1_ref)):
        out = (jnp.dot(hidden, woa[...], preferred_element_type=f32)
               + jnp.dot(ctx, wob[...], preferred_element_type=f32)
               + bo[...])
        x = hidden + out
        mu = jnp.mean(x, axis=1, keepdims=True)
        xc = x - mu
        var = jnp.mean(xc * xc, axis=1, keepdims=True)
        hidden = xc * lax.rsqrt(var + 1e-5) * g[...] + be[...]
    hidden_out_ref[...] = hidden


def _tc_forward(nbf_t3, hid_rows, nbw_rows, items_col, sess, pos_table,
                w1t, w1b, b1r, w2both, b2cat, exp_mat,
                woa0, wob0, bo0, g0, be0, woa1, wob1, bo1, g1, be1,
                interpret=False):
    full = lambda shape: pl.BlockSpec(shape, lambda i: tuple(0 for _ in shape))
    return pl.pallas_call(
        _tc_body,
        grid=(GRID,),
        in_specs=[
            pl.BlockSpec((N, R, D), lambda i: (0, i, 0)),
            pl.BlockSpec((R, D), lambda i: (i, 0)),
            pl.BlockSpec((R, N), lambda i: (i, 0)),
            pl.BlockSpec((R, 1), lambda i: (i, 0)),
            pl.BlockSpec((R, D), lambda i: (i, 0)),
            pl.BlockSpec((R, D), lambda i: (i, 0)),
            full((D, 2 * H)),
            full((D, 2 * H)),
            full((1, 2 * H)),
            full((N, 2 * H, 2 * N)),
            full((1, 2 * N)),
            full((N, N * D)),
            full((D, D)), full((D, D)), full((1, D)), full((1, D)), full((1, D)),
            full((D, D)), full((D, D)), full((1, D)), full((1, D)), full((1, D)),
        ],
        out_specs=[
            pl.BlockSpec((R, D), lambda i: (i, 0)),
            pl.BlockSpec((R, N), lambda i: (i, 0)),
            pl.BlockSpec((R, N), lambda i: (i, 0)),
        ],
        out_shape=[
            jax.ShapeDtypeStruct((ROWS, D), jnp.float32),
            jax.ShapeDtypeStruct((ROWS, N), jnp.float32),
            jax.ShapeDtypeStruct((ROWS, N), jnp.float32),
        ],
        interpret=interpret,
    )(nbf_t3, hid_rows, nbw_rows, items_col, sess, pos_table,
      w1t, w1b, b1r, w2both, b2cat, exp_mat,
      woa0, wob0, bo0, g0, be0, woa1, wob1, bo1, g1, be1)


def kernel(items, neighbor_table, neighbor_weight_table, session_embedding,
           item_table, pos_table,
           W1_0, b1_0, W2_0, b2_0, Wo_0, bo_0, g_0, be_0,
           W1_1, b1_1, W2_1, b2_1, Wo_1, bo_1, g_1, be_1):
    items_flat = items.reshape(ROWS).astype(jnp.int32)
    nbs_t_flat = neighbor_table[items_flat].T.reshape(N * ROWS)
    nbw_rows = neighbor_weight_table[items_flat]
    hid_rows, nbf_t = _sc_gather(items_flat, nbs_t_flat, item_table)
    nbf_t3 = nbf_t.reshape(N, ROWS, D)

    w1t = jnp.concatenate([W1_0[:D], W1_1[:D]], axis=1)
    w1b = jnp.concatenate([W1_0[D:], W1_1[D:]],
                          axis=1).astype(jnp.bfloat16)
    b1r = jnp.concatenate([b1_0, b1_1], axis=0).reshape(1, 2 * H)
    eye = jnp.eye(N, dtype=jnp.float32)
    z = jnp.zeros((N, H, N), jnp.float32)
    top = jnp.concatenate(
        [W2_0.reshape(1, H, 1) * eye[:, None, :], z], axis=2)
    bot = jnp.concatenate(
        [z, W2_1.reshape(1, H, 1) * eye[:, None, :]], axis=2)
    w2both = jnp.concatenate([top, bot], axis=1).astype(jnp.bfloat16)
    b2cat = jnp.concatenate(
        [jnp.broadcast_to(b2_0, (N,)), jnp.broadcast_to(b2_1, (N,))],
        axis=0).reshape(1, 2 * N)
    exp_mat = jnp.repeat(jnp.eye(N, dtype=jnp.float32), D, axis=1)

    sess_exp = jnp.broadcast_to(
        session_embedding[:, None, :], (B, S, D)).reshape(ROWS, D)
    pos_exp = jnp.broadcast_to(
        pos_table[None, :S, :], (B, S, D)).reshape(ROWS, D)
    hidden_f, aw0_f, aw1_f = _tc_forward(
        nbf_t3, hid_rows, nbw_rows, items_flat.reshape(ROWS, 1),
        sess_exp, pos_exp,
        w1t, w1b, b1r, w2both, b2cat, exp_mat,
        Wo_0[:D], Wo_0[D:], bo_0.reshape(1, D), g_0.reshape(1, D),
        be_0.reshape(1, D),
        Wo_1[:D], Wo_1[D:], bo_1.reshape(1, D), g_1.reshape(1, D),
        be_1.reshape(1, D))

    return (hidden_f.reshape(B, S, D),
            aw0_f.reshape(B, S, N),
            aw1_f.reshape(B, S, N))
